# baseline (speedup 1.0000x reference)
import functools

import jax
import jax.numpy as jnp
from jax import lax
from jax.experimental import pallas as pl
from jax.experimental.pallas import tpu as pltpu
from jax.experimental.pallas import tpu_sc as plsc

N = 10000
E = 320000
IN = 128
H = 256
OUT = 10

NC, NS = 2, 16
LANES = 128
NACC = 10240
RPT = NACC // NS
K1 = 80
K2 = 160
G = 16
RBLK = 512

@functools.cache
def _mesh():
  return plsc.VectorSubcoreMesh(
      core_axis_name="c", subcore_axis_name="s", num_cores=NC,
      num_subcores=NS)


def _deg_body(didx_hbm, zero_hbm, ones_hbm, deg_hbm, didx_v, ones_v, acc, sem):
  c = lax.axis_index("c")
  s = lax.axis_index("s")
  del sem
  pltpu.sync_copy(zero_hbm.at[pl.ds(s * RPT, RPT)], acc.at[pl.ds(s * RPT, RPT)])
  pltpu.sync_copy(didx_hbm.at[c, s], didx_v)
  pltpu.sync_copy(ones_hbm, ones_v)
  plsc.subcore_barrier()

  def step(j, _):
    pltpu.sync_copy(ones_v, acc.at[didx_v.at[j]], add=True)
    return 0
  lax.fori_loop(0, K1, step, 0)
  plsc.subcore_barrier()
  pltpu.sync_copy(acc.at[pl.ds(s * RPT, RPT)],
                  deg_hbm.at[c, pl.ds(s * RPT, RPT)])


@functools.cache
def _deg_call():
  return pl.kernel(
      _deg_body,
      out_type=jax.ShapeDtypeStruct((NC, NACC, LANES), jnp.float32),
      mesh=_mesh(),
      scratch_types=[
          pltpu.VMEM((K1, LANES), jnp.int32),
          pltpu.VMEM((LANES, LANES), jnp.float32),
          pltpu.VMEM_SHARED((NACC, LANES), jnp.float32),
          pltpu.SemaphoreType.DMA,
      ],
  )


def _make_agg(K):
  def body(table_hbm, sidx_hbm, didx_hbm, out_hbm,
           sidx_v, didx_v, rows0_v, rows1_v, acc, sem0, sem1):
    c = lax.axis_index("c")
    s = lax.axis_index("s")
    pltpu.sync_copy(table_hbm.at[pl.ds(c * NACC + s * RPT, RPT)],
                    acc.at[pl.ds(s * RPT, RPT)])
    plsc.subcore_barrier()

    rows = (rows0_v, rows1_v)
    sems = (sem0, sem1)

    def group(g, _):
      pltpu.sync_copy(sidx_hbm.at[c, s, pl.ds(g * G, G)], sidx_v)
      pltpu.sync_copy(didx_hbm.at[c, s, pl.ds(g * G, G)], didx_v)
      for b in range(2):
        pltpu.async_copy(table_hbm.at[sidx_v.at[b]], rows[b], sems[b])

      def rnd(j, _):
        for b in range(2):
          ch = j * 2 + b
          pltpu.make_async_copy(table_hbm.at[sidx_v.at[0]], rows[b],
                                sems[b]).wait()
          pltpu.sync_copy(rows[b], acc.at[didx_v.at[ch]], add=True)

          @pl.when(ch + 2 < G)
          def _():
            pltpu.async_copy(table_hbm.at[sidx_v.at[ch + 2]], rows[b],
                             sems[b])
        return 0
      lax.fori_loop(0, G // 2, rnd, 0)
      return 0
    lax.fori_loop(0, K // G, group, 0)
    plsc.subcore_barrier()
    pltpu.sync_copy(acc.at[pl.ds(s * RPT, RPT)],
                    out_hbm.at[c, pl.ds(s * RPT, RPT)])

  return pl.kernel(
      body,
      out_type=jax.ShapeDtypeStruct((NC, NACC, IN), jnp.float32),
      mesh=_mesh(),
      scratch_types=[
          pltpu.VMEM((G, LANES), jnp.int32),
          pltpu.VMEM((G, LANES), jnp.int32),
          pltpu.VMEM((LANES, IN), jnp.float32),
          pltpu.VMEM((LANES, IN), jnp.float32),
          pltpu.VMEM_SHARED((NACC, IN), jnp.float32),
          pltpu.SemaphoreType.DMA,
          pltpu.SemaphoreType.DMA,
      ],
  )


_make_agg = functools.cache(_make_agg)


def _dinv_block(degp):
  d = degp[0, :, 0:1] + degp[1, :, 0:1] + 1.0
  return lax.rsqrt(d)


def _prep_body(degp_ref, x_ref, out_ref):
  dinv = _dinv_block(degp_ref[...])
  out_ref[0] = x_ref[...] * dinv
  out_ref[1] = jnp.zeros_like(x_ref[...])


_prep_call = pl.pallas_call(
    _prep_body,
    grid=(NACC // RBLK,),
    in_specs=[
        pl.BlockSpec((NC, RBLK, LANES), lambda i: (0, i, 0)),
        pl.BlockSpec((RBLK, IN), lambda i: (i, 0)),
    ],
    out_specs=pl.BlockSpec((NC, RBLK, IN), lambda i: (0, i, 0)),
    out_shape=jax.ShapeDtypeStruct((NC, NACC, IN), jnp.float32),
)


def _layer_body(mode, last, p_ref, degp_ref, w_ref, b_ref, out_ref):
  dinv = _dinv_block(degp_ref[...])
  if mode == "sum":
    m = (p_ref[0] + p_ref[1]) * dinv
  else:
    m = jnp.concatenate([p_ref[0], p_ref[1]], axis=1) * dinv
  h = jnp.maximum(
      jnp.dot(m, w_ref[...], preferred_element_type=jnp.float32) + b_ref[...],
      0.0)
  if last:
    out_ref[...] = h
  else:
    th = h * dinv
    out_ref[0] = th[:, :IN]
    out_ref[1] = th[:, IN:]


def _make_layer(mode, last, fin):
  out_spec = (pl.BlockSpec((RBLK, H), lambda i: (i, 0)) if last
              else pl.BlockSpec((NC, RBLK, IN), lambda i: (0, i, 0)))
  out_shape = (jax.ShapeDtypeStruct((NACC, H), jnp.float32) if last
               else jax.ShapeDtypeStruct((NC, NACC, IN), jnp.float32))
  return pl.pallas_call(
      functools.partial(_layer_body, mode, last),
      grid=(NACC // RBLK,),
      in_specs=[
          pl.BlockSpec((NC, RBLK, IN), lambda i: (0, i, 0)),
          pl.BlockSpec((NC, RBLK, LANES), lambda i: (0, i, 0)),
          pl.BlockSpec((fin, H), lambda i: (0, 0)),
          pl.BlockSpec((1, H), lambda i: (0, 0)),
      ],
      out_specs=out_spec,
      out_shape=out_shape,
  )


_layer1 = _make_layer("sum", False, IN)
_layer2 = _make_layer("cat", False, H)
_layer3 = _make_layer("cat", True, H)


def _head_body(h_ref, wm1_ref, bm1_ref, wm2_ref, bm2_ref, out_ref, acc_ref):
  i = pl.program_id(0)

  @pl.when(i == 0)
  def _():
    acc_ref[...] = jnp.zeros_like(acc_ref)

  rows = i * RBLK + lax.broadcasted_iota(jnp.int32, (RBLK, 1), 0)
  hm = jnp.where(rows < N, h_ref[...], 0.0)
  acc_ref[...] += hm.reshape(RBLK // 8, 8, H).sum(axis=0)

  @pl.when(i == pl.num_programs(0) - 1)
  def _():
    g = acc_ref[...].sum(axis=0, keepdims=True) * (1.0 / N)
    z = jnp.maximum(
        jnp.dot(g, wm1_ref[...], preferred_element_type=jnp.float32)
        + bm1_ref[...], 0.0)
    out_ref[...] = (
        jnp.dot(z, wm2_ref[...], preferred_element_type=jnp.float32)
        + bm2_ref[...])


_head_call = pl.pallas_call(
    _head_body,
    grid=(NACC // RBLK,),
    in_specs=[
        pl.BlockSpec((RBLK, H), lambda i: (i, 0)),
        pl.BlockSpec((H, H), lambda i: (0, 0)),
        pl.BlockSpec((1, H), lambda i: (0, 0)),
        pl.BlockSpec((H, OUT), lambda i: (0, 0)),
        pl.BlockSpec((1, OUT), lambda i: (0, 0)),
    ],
    out_specs=pl.BlockSpec((1, OUT), lambda i: (0, 0)),
    out_shape=jax.ShapeDtypeStruct((1, OUT), jnp.float32),
    scratch_shapes=[pltpu.VMEM((8, H), jnp.float32)],
)


def kernel(x, edge_index, W1, b1, W2, b2, W3, b3, Wm1, bm1, Wm2, bm2):
  src = edge_index[0]
  dst = edge_index[1]

  x_pad = jnp.zeros((NACC, IN), jnp.float32).at[:N].set(x)

  ec = E // 2
  per1 = NS * K1 * LANES
  pad1 = jnp.full((NC, per1 - ec), N, jnp.int32)
  src1 = jnp.concatenate([src.reshape(NC, ec), pad1], axis=1)
  src1 = src1.reshape(NC, NS, K1, LANES)
  dst1 = jnp.concatenate([dst.reshape(NC, ec), pad1], axis=1)
  dst1 = dst1.reshape(NC, NS, K1, LANES)

  per2 = NS * K2 * LANES
  pad2 = jnp.full((per2 - E,), N, jnp.int32)
  srcp = jnp.concatenate([src, pad2]).reshape(NS, K2, LANES)
  dstp = jnp.concatenate([dst, pad2]).reshape(NS, K2, LANES)
  sidx2 = jnp.stack([srcp, srcp + NACC])
  didx2 = jnp.stack([dstp, dstp])

  agg_es = _make_agg(K1)
  agg_fs = _make_agg(K2)

  zerosw = jnp.zeros((NACC, LANES), jnp.float32)
  onesw = jnp.ones((LANES, LANES), jnp.float32)
  degp = _deg_call()(dst1, zerosw, onesw)

  t0pair = _prep_call(degp, x_pad)
  p1 = agg_es(t0pair.reshape(NC * NACC, IN), src1, dst1)
  t1pair = _layer1(p1, degp, W1, b1.reshape(1, H))
  p2 = agg_fs(t1pair.reshape(NC * NACC, IN), sidx2, didx2)
  t2pair = _layer2(p2, degp, W2, b2.reshape(1, H))
  p3 = agg_fs(t2pair.reshape(NC * NACC, IN), sidx2, didx2)
  h3 = _layer3(p3, degp, W3, b3.reshape(1, H))
  out = _head_call(h3, Wm1, bm1.reshape(1, H), Wm2, bm2.reshape(1, OUT))
  return out[0]


# speedup vs baseline: 1.1202x; 1.1202x over previous
import functools

import jax
import jax.numpy as jnp
from jax import lax
from jax.experimental import pallas as pl
from jax.experimental.pallas import tpu as pltpu
from jax.experimental.pallas import tpu_sc as plsc

N = 10000
E = 320000
IN = 128
H = 256
OUT = 10

NC, NS = 2, 16
LANES = 128
NACC = 10240
RPT = NACC // NS
K1 = 80
K2 = 160
G = 32
RBLK = 512

@functools.cache
def _mesh():
  return plsc.VectorSubcoreMesh(
      core_axis_name="c", subcore_axis_name="s", num_cores=NC,
      num_subcores=NS)


def _deg_body(didx_hbm, zero_hbm, ones_hbm, deg_hbm, didx_v, ones_v, acc, sem):
  c = lax.axis_index("c")
  s = lax.axis_index("s")
  del sem
  pltpu.sync_copy(zero_hbm.at[pl.ds(s * RPT, RPT)], acc.at[pl.ds(s * RPT, RPT)])
  pltpu.sync_copy(didx_hbm.at[c, s], didx_v)
  pltpu.sync_copy(ones_hbm, ones_v)
  plsc.subcore_barrier()

  def step(j, _):
    pltpu.sync_copy(ones_v, acc.at[didx_v.at[j]], add=True)
    return 0
  lax.fori_loop(0, K1, step, 0)
  plsc.subcore_barrier()
  pltpu.sync_copy(acc.at[pl.ds(s * RPT, RPT)],
                  deg_hbm.at[c, pl.ds(s * RPT, RPT)])


@functools.cache
def _deg_call():
  return pl.kernel(
      _deg_body,
      out_type=jax.ShapeDtypeStruct((NC, NACC, LANES), jnp.float32),
      mesh=_mesh(),
      scratch_types=[
          pltpu.VMEM((K1, LANES), jnp.int32),
          pltpu.VMEM((LANES, LANES), jnp.float32),
          pltpu.VMEM_SHARED((NACC, LANES), jnp.float32),
          pltpu.SemaphoreType.DMA,
      ],
  )


def _make_agg(K):
  def body(table_hbm, sidx_hbm, didx_hbm, out_hbm,
           sidx_v, didx_v, rows0_v, rows1_v, acc, sem0, sem1):
    c = lax.axis_index("c")
    s = lax.axis_index("s")
    pltpu.sync_copy(table_hbm.at[pl.ds(c * NACC + s * RPT, RPT)],
                    acc.at[pl.ds(s * RPT, RPT)])
    plsc.subcore_barrier()

    rows = (rows0_v, rows1_v)
    sems = (sem0, sem1)

    def group(g, _):
      pltpu.sync_copy(sidx_hbm.at[c, s, pl.ds(g * G, G)], sidx_v)
      pltpu.sync_copy(didx_hbm.at[c, s, pl.ds(g * G, G)], didx_v)
      for b in range(2):
        pltpu.async_copy(table_hbm.at[sidx_v.at[b]], rows[b], sems[b])

      def rnd(j, _):
        for b in range(2):
          ch = j * 2 + b
          pltpu.make_async_copy(table_hbm.at[sidx_v.at[0]], rows[b],
                                sems[b]).wait()
          pltpu.sync_copy(rows[b], acc.at[didx_v.at[ch]], add=True)

          @pl.when(ch + 2 < G)
          def _():
            pltpu.async_copy(table_hbm.at[sidx_v.at[ch + 2]], rows[b],
                             sems[b])
        return 0
      lax.fori_loop(0, G // 2, rnd, 0)
      return 0
    lax.fori_loop(0, K // G, group, 0)
    plsc.subcore_barrier()
    pltpu.sync_copy(acc.at[pl.ds(s * RPT, RPT)],
                    out_hbm.at[c, pl.ds(s * RPT, RPT)])

  return pl.kernel(
      body,
      out_type=jax.ShapeDtypeStruct((NC, NACC, IN), jnp.float32),
      mesh=_mesh(),
      scratch_types=[
          pltpu.VMEM((G, LANES), jnp.int32),
          pltpu.VMEM((G, LANES), jnp.int32),
          pltpu.VMEM((LANES, IN), jnp.float32),
          pltpu.VMEM((LANES, IN), jnp.float32),
          pltpu.VMEM_SHARED((NACC, IN), jnp.float32),
          pltpu.SemaphoreType.DMA,
          pltpu.SemaphoreType.DMA,
      ],
  )


_make_agg = functools.cache(_make_agg)


def _dinv_block(degp):
  d = degp[0, :, 0:1] + degp[1, :, 0:1] + 1.0
  return lax.rsqrt(d)


def _prep_body(degp_ref, x_ref, out_ref):
  dinv = _dinv_block(degp_ref[...])
  out_ref[0] = x_ref[...] * dinv
  out_ref[1] = jnp.zeros_like(x_ref[...])


_prep_call = pl.pallas_call(
    _prep_body,
    grid=(NACC // RBLK,),
    in_specs=[
        pl.BlockSpec((NC, RBLK, LANES), lambda i: (0, i, 0)),
        pl.BlockSpec((RBLK, IN), lambda i: (i, 0)),
    ],
    out_specs=pl.BlockSpec((NC, RBLK, IN), lambda i: (0, i, 0)),
    out_shape=jax.ShapeDtypeStruct((NC, NACC, IN), jnp.float32),
)


def _layer_body(mode, last, p_ref, degp_ref, w_ref, b_ref, out_ref):
  dinv = _dinv_block(degp_ref[...])
  if mode == "sum":
    m = (p_ref[0] + p_ref[1]) * dinv
  else:
    m = jnp.concatenate([p_ref[0], p_ref[1]], axis=1) * dinv
  h = jnp.maximum(
      jnp.dot(m, w_ref[...], preferred_element_type=jnp.float32) + b_ref[...],
      0.0)
  if last:
    out_ref[...] = h
  else:
    th = h * dinv
    out_ref[0] = th[:, :IN]
    out_ref[1] = th[:, IN:]


def _make_layer(mode, last, fin):
  out_spec = (pl.BlockSpec((RBLK, H), lambda i: (i, 0)) if last
              else pl.BlockSpec((NC, RBLK, IN), lambda i: (0, i, 0)))
  out_shape = (jax.ShapeDtypeStruct((NACC, H), jnp.float32) if last
               else jax.ShapeDtypeStruct((NC, NACC, IN), jnp.float32))
  return pl.pallas_call(
      functools.partial(_layer_body, mode, last),
      grid=(NACC // RBLK,),
      in_specs=[
          pl.BlockSpec((NC, RBLK, IN), lambda i: (0, i, 0)),
          pl.BlockSpec((NC, RBLK, LANES), lambda i: (0, i, 0)),
          pl.BlockSpec((fin, H), lambda i: (0, 0)),
          pl.BlockSpec((1, H), lambda i: (0, 0)),
      ],
      out_specs=out_spec,
      out_shape=out_shape,
  )


_layer1 = _make_layer("sum", False, IN)
_layer2 = _make_layer("cat", False, H)
_layer3 = _make_layer("cat", True, H)


def _head_body(h_ref, wm1_ref, bm1_ref, wm2_ref, bm2_ref, out_ref, acc_ref):
  i = pl.program_id(0)

  @pl.when(i == 0)
  def _():
    acc_ref[...] = jnp.zeros_like(acc_ref)

  rows = i * RBLK + lax.broadcasted_iota(jnp.int32, (RBLK, 1), 0)
  hm = jnp.where(rows < N, h_ref[...], 0.0)
  acc_ref[...] += hm.reshape(RBLK // 8, 8, H).sum(axis=0)

  @pl.when(i == pl.num_programs(0) - 1)
  def _():
    g = acc_ref[...].sum(axis=0, keepdims=True) * (1.0 / N)
    z = jnp.maximum(
        jnp.dot(g, wm1_ref[...], preferred_element_type=jnp.float32)
        + bm1_ref[...], 0.0)
    out_ref[...] = (
        jnp.dot(z, wm2_ref[...], preferred_element_type=jnp.float32)
        + bm2_ref[...])


_head_call = pl.pallas_call(
    _head_body,
    grid=(NACC // RBLK,),
    in_specs=[
        pl.BlockSpec((RBLK, H), lambda i: (i, 0)),
        pl.BlockSpec((H, H), lambda i: (0, 0)),
        pl.BlockSpec((1, H), lambda i: (0, 0)),
        pl.BlockSpec((H, OUT), lambda i: (0, 0)),
        pl.BlockSpec((1, OUT), lambda i: (0, 0)),
    ],
    out_specs=pl.BlockSpec((1, OUT), lambda i: (0, 0)),
    out_shape=jax.ShapeDtypeStruct((1, OUT), jnp.float32),
    scratch_shapes=[pltpu.VMEM((8, H), jnp.float32)],
)


def kernel(x, edge_index, W1, b1, W2, b2, W3, b3, Wm1, bm1, Wm2, bm2):
  src = edge_index[0]
  dst = edge_index[1]

  x_pad = jnp.zeros((NACC, IN), jnp.float32).at[:N].set(x)

  ec = E // 2
  per1 = NS * K1 * LANES
  pad1 = jnp.full((NC, per1 - ec), N, jnp.int32)
  src1 = jnp.concatenate([src.reshape(NC, ec), pad1], axis=1)
  src1 = src1.reshape(NC, NS, K1, LANES)
  dst1 = jnp.concatenate([dst.reshape(NC, ec), pad1], axis=1)
  dst1 = dst1.reshape(NC, NS, K1, LANES)

  per2 = NS * K2 * LANES
  pad2 = jnp.full((per2 - E,), N, jnp.int32)
  srcp = jnp.concatenate([src, pad2]).reshape(NS, K2, LANES)
  dstp = jnp.concatenate([dst, pad2]).reshape(NS, K2, LANES)
  sidx2 = jnp.stack([srcp, srcp + NACC])
  didx2 = jnp.stack([dstp, dstp])

  agg_es = _make_agg(K1)
  agg_fs = _make_agg(K2)

  zerosw = jnp.zeros((NACC, LANES), jnp.float32)
  onesw = jnp.ones((LANES, LANES), jnp.float32)
  degp = _deg_call()(dst1, zerosw, onesw)

  t0pair = _prep_call(degp, x_pad)
  p1 = agg_es(t0pair.reshape(NC * NACC, IN), src1, dst1)
  t1pair = _layer1(p1, degp, W1, b1.reshape(1, H))
  p2 = agg_fs(t1pair.reshape(NC * NACC, IN), sidx2, didx2)
  t2pair = _layer2(p2, degp, W2, b2.reshape(1, H))
  p3 = agg_fs(t2pair.reshape(NC * NACC, IN), sidx2, didx2)
  h3 = _layer3(p3, degp, W3, b3.reshape(1, H))
  out = _head_call(h3, Wm1, bm1.reshape(1, H), Wm2, bm2.reshape(1, OUT))
  return out[0]
